# baseline (speedup 1.0000x reference)
import functools

import numpy as np
import jax
import jax.numpy as jnp
from jax import lax
from jax.experimental import pallas as pl
from jax.experimental.pallas import tpu as pltpu
from jax.experimental.pallas import tpu_sc as plsc

N_NODES = 10000
N_EDGES = 320000
D_FEAT = 128
ALPHA = 0.5
HOP = 3

NC = 2
NS = 16
NW = NC * NS
EPW = N_EDGES // NW
K = 40
NCHUNK = EPW // K
ZCH = N_NODES // K
UNROLL = 6

_PERM = (
    np.arange(D_FEAT).reshape(4, 2, 16).transpose(0, 2, 1).reshape(D_FEAT)
)


def _scatter_body(cf_hbm, vals_hbm, meta_hbm, out_hbm,
                  meta0, meta1, meta2, vrep0, vrep1,
                  gb0, gb1, sb0, sb1, acc,
                  semm0, semm1, semm2, semv0, semv1,
                  semg0, semg1, sems0, sems1):
    metas = (meta0, meta1, meta2)
    vreps = (vrep0, vrep1)
    gbufs = (gb0, gb1)
    sbufs = (sb0, sb1)
    semms = (semm0, semm1, semm2)
    semvs = (semv0, semv1)
    semgs = (semg0, semg1)
    semss = (sems0, sems1)

    c = lax.axis_index("c")
    s = lax.axis_index("s")
    wid = c * NS + s

    def fire_meta(i, t):
        pltpu.async_copy(meta_hbm.at[wid, i], metas[t], semms[t])

    def wait_meta(t):
        pltpu.make_async_copy(meta_hbm.at[wid, 0], metas[t], semms[t]).wait()

    def fire_vrep(i, b):
        pltpu.async_copy(vals_hbm.at[wid, i], vreps[b], semvs[b])

    def wait_vrep(b):
        pltpu.make_async_copy(vals_hbm.at[wid, 0], vreps[b], semvs[b]).wait()

    def fire_gather(b, t):
        pltpu.async_copy(cf_hbm.at[metas[t].at[0]], gbufs[b], semgs[b])

    def wait_gather(b, t):
        pltpu.make_async_copy(
            cf_hbm.at[metas[t].at[0]], gbufs[b], semgs[b]).wait()

    def fire_scatter(b, t):
        pltpu.async_copy(sbufs[b], acc.at[metas[t].at[1]], semss[b], add=True)

    def wait_scatter(b, t):
        pltpu.make_async_copy(sbufs[b], acc.at[metas[t].at[1]], semss[b]).wait()

    zero16 = jnp.zeros((16,), jnp.float32)

    def zero_row(i, _):
        for r8 in range(8):
            sb0[i, pl.ds(r8 * 16, 16)] = zero16
        return 0
    lax.fori_loop(0, K, zero_row, 0)
    for u in range(16):
        j = s + u * NS
        @pl.when(j < ZCH)
        def _():
            pltpu.sync_copy(sb0, acc.at[pl.ds(j * K, K)])
    plsc.subcore_barrier()

    fire_meta(0, 0)
    fire_meta(1, 1)
    fire_vrep(0, 0)
    fire_vrep(1, 1)
    wait_meta(0)
    fire_gather(0, 0)

    def sub(i, b, t):
        bn = 1 - b
        wait_gather(b, t)

        @pl.when(i + 1 < NCHUNK)
        def _():
            @pl.when(i >= 1)
            def _():
                wait_scatter(bn, (t + 2) % 3)
            wait_meta((t + 1) % 3)

            @pl.when(i + 2 < NCHUNK)
            def _():
                fire_meta(i + 2, (t + 2) % 3)
            fire_gather(bn, (t + 1) % 3)

        wait_vrep(b)

        sh16 = jnp.full((16,), 16, jnp.int32)

        @plsc.parallel_loop(0, K, step=1, unroll=4)
        def _(j):
            v16 = vreps[b][j]
            for q in range(4):
                w = gbufs[b][j, pl.ds(q * 16, 16)]
                lo_i = lax.shift_right_arithmetic(lax.shift_left(w, sh16), sh16)
                hi_i = lax.shift_right_arithmetic(w, sh16)
                lo = lax.convert_element_type(lo_i, jnp.float32)
                hi = lax.convert_element_type(hi_i, jnp.float32)
                sbufs[b][j, pl.ds(q * 32, 16)] = lo * v16
                sbufs[b][j, pl.ds(q * 32 + 16, 16)] = hi * v16

        fire_scatter(b, t)

        @pl.when(i + 2 < NCHUNK)
        def _():
            fire_vrep(i + 2, b)

    def outer(io, _):
        for p in range(UNROLL):
            i = io * UNROLL + p
            @pl.when(i < NCHUNK)
            def _():
                sub(i, p % 2, p % 3)
        return 0
    lax.fori_loop(0, (NCHUNK + UNROLL - 1) // UNROLL, outer, 0)

    wait_scatter((NCHUNK - 2) % 2, (NCHUNK - 2) % 3)
    wait_scatter((NCHUNK - 1) % 2, (NCHUNK - 1) % 3)
    plsc.subcore_barrier()

    for u in range(16):
        j = s + u * NS
        @pl.when(j < ZCH)
        def _():
            pltpu.sync_copy(acc.at[pl.ds(j * K, K)],
                            out_hbm.at[c, pl.ds(j * K, K)])


_scatter = pl.kernel(
    _scatter_body,
    out_type=jax.ShapeDtypeStruct((NC, N_NODES, D_FEAT), jnp.float32),
    mesh=plsc.VectorSubcoreMesh(core_axis_name="c", subcore_axis_name="s"),
    compiler_params=pltpu.CompilerParams(use_tc_tiling_on_sc=False),
    scratch_types=[
        pltpu.VMEM((2, K), jnp.int32),
        pltpu.VMEM((2, K), jnp.int32),
        pltpu.VMEM((2, K), jnp.int32),
        pltpu.VMEM((K, 16), jnp.float32),
        pltpu.VMEM((K, 16), jnp.float32),
        pltpu.VMEM((K, D_FEAT // 2), jnp.int32),
        pltpu.VMEM((K, D_FEAT // 2), jnp.int32),
        pltpu.VMEM((K, D_FEAT), jnp.float32),
        pltpu.VMEM((K, D_FEAT), jnp.float32),
        pltpu.VMEM_SHARED((N_NODES, D_FEAT), jnp.float32),
        pltpu.SemaphoreType.DMA,
        pltpu.SemaphoreType.DMA,
        pltpu.SemaphoreType.DMA,
        pltpu.SemaphoreType.DMA,
        pltpu.SemaphoreType.DMA,
        pltpu.SemaphoreType.DMA,
        pltpu.SemaphoreType.DMA,
        pltpu.SemaphoreType.DMA,
        pltpu.SemaphoreType.DMA,
    ],
)


def _combine_body(sc_ref, ac_ref, c_ref, o_ref):
    o_ref[...] = sc_ref[0, 0] * (ac_ref[0] + ac_ref[1]) + c_ref[...]


_BR = 400

_combine = pl.pallas_call(
    _combine_body,
    out_shape=jax.ShapeDtypeStruct((N_NODES, D_FEAT), jnp.float32),
    grid=(N_NODES // _BR,),
    in_specs=[
        pl.BlockSpec((1, 1), lambda i: (0, 0)),
        pl.BlockSpec((NC, _BR, D_FEAT), lambda i: (0, i, 0)),
        pl.BlockSpec((_BR, D_FEAT), lambda i: (i, 0)),
    ],
    out_specs=pl.BlockSpec((_BR, D_FEAT), lambda i: (i, 0)),
)


def kernel(C, vals, row, col):
    vals_r = jnp.broadcast_to(
        vals.reshape(NW, NCHUNK, K, 1), (NW, NCHUNK, K, 16)
    )
    meta = jnp.stack(
        [col.reshape(NW, NCHUNK, K), row.reshape(NW, NCHUNK, K)], axis=2
    )
    perm = jnp.asarray(_PERM)
    cf = C
    for _ in range(HOP):
        scale = jnp.maximum(jnp.max(jnp.abs(cf)), 1e-30)
        q16 = jnp.clip(
            jnp.round(cf[:, perm] * (32000.0 / scale)), -32767.0, 32767.0
        ).astype(jnp.int16)
        cfp = jax.lax.bitcast_convert_type(
            q16.reshape(N_NODES, D_FEAT // 2, 2), jnp.int32
        )
        ac = _scatter(cfp, vals_r, meta)
        sc = (ALPHA * scale / 32000.0).reshape(1, 1)
        cf = _combine(sc, ac, C)
    return cf


# speedup vs baseline: 1.6277x; 1.6277x over previous
import functools

import jax
import jax.numpy as jnp
from jax import lax
from jax.experimental import pallas as pl
from jax.experimental.pallas import tpu as pltpu
from jax.experimental.pallas import tpu_sc as plsc

N_NODES = 10000
N_EDGES = 320000
D_FEAT = 128
ALPHA = 0.5
HOP = 3

NC = 2
NS = 16
NW = NC * NS
EPW = N_EDGES // NW
K = 80
NCHUNK = EPW // K
ZCH = N_NODES // K
UNROLL = 6


def _scatter_body(cf_hbm, vals_hbm, meta_hbm, out_hbm,
                  meta0, meta1, meta2, vrep0, vrep1,
                  ga0, gb0, ga1, gb1, acc,
                  semm0, semm1, semm2, semv0, semv1,
                  semga0, semgb0, semga1, semgb1,
                  semsa0, semsb0, semsa1, semsb1):
    metas = (meta0, meta1, meta2)
    vreps = (vrep0, vrep1)
    gbufs = ((ga0, gb0), (ga1, gb1))
    semms = (semm0, semm1, semm2)
    semvs = (semv0, semv1)
    semgs = ((semga0, semgb0), (semga1, semgb1))
    semss = ((semsa0, semsb0), (semsa1, semsb1))
    KH = K // 2

    c = lax.axis_index("c")
    s = lax.axis_index("s")
    wid = c * NS + s

    def fire_meta(i, t):
        pltpu.async_copy(meta_hbm.at[wid, i], metas[t], semms[t])

    def wait_meta(t):
        pltpu.make_async_copy(meta_hbm.at[wid, 0], metas[t], semms[t]).wait()

    def fire_vrep(i, b):
        pltpu.async_copy(vals_hbm.at[wid, i], vreps[b], semvs[b])

    def wait_vrep(b):
        pltpu.make_async_copy(vals_hbm.at[wid, 0], vreps[b], semvs[b]).wait()

    def fire_gather(b, t):
        for h in range(2):
            pltpu.async_copy(
                cf_hbm.at[metas[t].at[0, pl.ds(h * KH, KH)]],
                gbufs[b][h], semgs[b][h])

    def wait_gather(b, t):
        for h in range(2):
            pltpu.make_async_copy(
                cf_hbm.at[metas[t].at[0, pl.ds(h * KH, KH)]],
                gbufs[b][h], semgs[b][h]).wait()

    def fire_scatter(b, t):
        for h in range(2):
            pltpu.async_copy(
                gbufs[b][h], acc.at[metas[t].at[1, pl.ds(h * KH, KH)]],
                semss[b][h], add=True)

    def wait_scatter(b, t):
        for h in range(2):
            pltpu.make_async_copy(
                gbufs[b][h], acc.at[metas[t].at[1, pl.ds(h * KH, KH)]],
                semss[b][h]).wait()

    zero16 = jnp.zeros((16,), jnp.float32)

    def zero_row(i, _):
        for r8 in range(8):
            ga0[i, pl.ds(r8 * 16, 16)] = zero16
            gb0[i, pl.ds(r8 * 16, 16)] = zero16
        return 0
    lax.fori_loop(0, KH, zero_row, 0)
    for u in range(8):
        j = s + u * NS
        @pl.when(j < ZCH)
        def _():
            pltpu.sync_copy(ga0, acc.at[pl.ds(j * K, KH)])
            pltpu.sync_copy(gb0, acc.at[pl.ds(j * K + KH, KH)])
    plsc.subcore_barrier()

    fire_meta(0, 0)
    fire_meta(1, 1)
    fire_vrep(0, 0)
    fire_vrep(1, 1)
    wait_meta(0)
    fire_gather(0, 0)

    def sub(i, b, t):
        bn = 1 - b
        wait_gather(b, t)

        @pl.when(i + 1 < NCHUNK)
        def _():
            @pl.when(i >= 1)
            def _():
                wait_scatter(bn, (t + 2) % 3)
            wait_meta((t + 1) % 3)

            @pl.when(i + 2 < NCHUNK)
            def _():
                fire_meta(i + 2, (t + 2) % 3)
            fire_gather(bn, (t + 1) % 3)

        wait_vrep(b)

        for h in range(2):
            gh = gbufs[b][h]

            @plsc.parallel_loop(0, KH, step=1, unroll=4)
            def _(j):
                v16 = vreps[b][j + h * KH]
                for r8 in range(8):
                    sl = pl.ds(r8 * 16, 16)
                    gh[j, sl] = gh[j, sl] * v16

        fire_scatter(b, t)

        @pl.when(i + 2 < NCHUNK)
        def _():
            fire_vrep(i + 2, b)

    def outer(io, _):
        for p in range(UNROLL):
            i = io * UNROLL + p
            @pl.when(i < NCHUNK)
            def _():
                sub(i, p % 2, p % 3)
        return 0
    lax.fori_loop(0, (NCHUNK + UNROLL - 1) // UNROLL, outer, 0)

    wait_scatter((NCHUNK - 2) % 2, (NCHUNK - 2) % 3)
    wait_scatter((NCHUNK - 1) % 2, (NCHUNK - 1) % 3)
    plsc.subcore_barrier()

    for u in range(8):
        j = s + u * NS
        @pl.when(j < ZCH)
        def _():
            pltpu.sync_copy(acc.at[pl.ds(j * K, K)],
                            out_hbm.at[c, pl.ds(j * K, K)])


_scatter = pl.kernel(
    _scatter_body,
    out_type=jax.ShapeDtypeStruct((NC, N_NODES, D_FEAT), jnp.float32),
    mesh=plsc.VectorSubcoreMesh(core_axis_name="c", subcore_axis_name="s"),
    scratch_types=[
        pltpu.VMEM((2, K), jnp.int32),
        pltpu.VMEM((2, K), jnp.int32),
        pltpu.VMEM((2, K), jnp.int32),
        pltpu.VMEM((K, 16), jnp.float32),
        pltpu.VMEM((K, 16), jnp.float32),
        pltpu.VMEM((K // 2, D_FEAT), jnp.float32),
        pltpu.VMEM((K // 2, D_FEAT), jnp.float32),
        pltpu.VMEM((K // 2, D_FEAT), jnp.float32),
        pltpu.VMEM((K // 2, D_FEAT), jnp.float32),
        pltpu.VMEM_SHARED((N_NODES, D_FEAT), jnp.float32),
        pltpu.SemaphoreType.DMA,
        pltpu.SemaphoreType.DMA,
        pltpu.SemaphoreType.DMA,
        pltpu.SemaphoreType.DMA,
        pltpu.SemaphoreType.DMA,
        pltpu.SemaphoreType.DMA,
        pltpu.SemaphoreType.DMA,
        pltpu.SemaphoreType.DMA,
        pltpu.SemaphoreType.DMA,
        pltpu.SemaphoreType.DMA,
        pltpu.SemaphoreType.DMA,
        pltpu.SemaphoreType.DMA,
        pltpu.SemaphoreType.DMA,
    ],
)


def _combine_body(ac_ref, c_ref, o_ref):
    o_ref[...] = ALPHA * (ac_ref[0] + ac_ref[1]) + c_ref[...]


_BR = 400

_combine = pl.pallas_call(
    _combine_body,
    out_shape=jax.ShapeDtypeStruct((N_NODES, D_FEAT), jnp.float32),
    grid=(N_NODES // _BR,),
    in_specs=[
        pl.BlockSpec((NC, _BR, D_FEAT), lambda i: (0, i, 0)),
        pl.BlockSpec((_BR, D_FEAT), lambda i: (i, 0)),
    ],
    out_specs=pl.BlockSpec((_BR, D_FEAT), lambda i: (i, 0)),
)


def kernel(C, vals, row, col):
    vals_r = jnp.broadcast_to(
        vals.reshape(NW, NCHUNK, K, 1), (NW, NCHUNK, K, 16)
    )
    meta = jnp.stack(
        [col.reshape(NW, NCHUNK, K), row.reshape(NW, NCHUNK, K)], axis=2
    )
    cf = C
    for _ in range(HOP):
        ac = _scatter(cf, vals_r, meta)
        cf = _combine(ac, C)
    return cf
